# Initial kernel scaffold; baseline (speedup 1.0000x reference)
import jax
import jax.numpy as jnp
from jax.experimental import pallas as pl


def kernel(lp_query_doc_features, tables, W1, b1, W2, b2):
    raise NotImplementedError("write your pallas kernel here")


# speedup vs baseline: 9.6055x; 9.6055x over previous
import functools

import jax
import jax.numpy as jnp
from jax import lax
from jax.experimental import pallas as pl
from jax.experimental.pallas import tpu as pltpu
from jax.experimental.pallas import tpu_sc as plsc

F, EDIM, HID = 26, 16, 32
VOCAB = 100000
NC, NS = 2, 16
NW = NC * NS
CHUNK = 128
G = 10


def _sc_gather(idx1, tbl2, n_rows):
    n_chunks = idx1.shape[0] // CHUNK
    chunks_per_w = n_chunks // NW
    groups_per_w = chunks_per_w // G
    mesh = plsc.VectorSubcoreMesh(core_axis_name="c", subcore_axis_name="s",
                                  num_cores=NC, num_subcores=NS)

    @functools.partial(
        pl.kernel,
        out_type=jax.ShapeDtypeStruct((n_rows, EDIM), jnp.float32),
        mesh=mesh,
        scratch_types=[
            pltpu.VMEM((G * CHUNK,), jnp.int32),
            pltpu.VMEM((G * CHUNK, EDIM), jnp.float32),
            pltpu.SemaphoreType.DMA,
        ],
        compiler_params=pltpu.CompilerParams(use_tc_tiling_on_sc=False),
    )
    def body(idx_hbm, tbl_hbm, out_hbm, idx_v, rows_v, sem):
        wid = lax.axis_index("s") * NC + lax.axis_index("c")
        cbase = wid * chunks_per_w

        def task(g, carry):
            rb = (cbase + g * G) * CHUNK
            pltpu.sync_copy(idx_hbm.at[pl.ds(rb, G * CHUNK)], idx_v)
            cps = [
                pltpu.async_copy(
                    tbl_hbm.at[idx_v.at[pl.ds(j * CHUNK, CHUNK)]],
                    rows_v.at[pl.ds(j * CHUNK, CHUNK)],
                    sem,
                )
                for j in range(G)
            ]
            for cp in cps:
                cp.wait()
            pltpu.sync_copy(rows_v, out_hbm.at[pl.ds(rb, G * CHUNK)])
            return carry

        lax.fori_loop(0, groups_per_w, task, 0)

    return body(idx1, tbl2)


def _mlp(emb2d, W1, b1, W2, b2, block_r):
    n = emb2d.shape[0]
    k = emb2d.shape[1]

    def body(emb_ref, w1_ref, b1_ref, w2_ref, b2_ref, out_ref):
        h = jnp.dot(emb_ref[...], w1_ref[...],
                    preferred_element_type=jnp.float32) + b1_ref[...]
        h = jnp.where(h > 0, h, jnp.exp(h) - 1.0)
        out_ref[...] = jnp.dot(h, w2_ref[...],
                               preferred_element_type=jnp.float32) + b2_ref[...]

    grid = (n // block_r,)
    return pl.pallas_call(
        body,
        grid=grid,
        in_specs=[
            pl.BlockSpec((block_r, k), lambda i: (i, 0)),
            pl.BlockSpec((k, HID), lambda i: (0, 0)),
            pl.BlockSpec((1, HID), lambda i: (0, 0)),
            pl.BlockSpec((HID, 1), lambda i: (0, 0)),
            pl.BlockSpec((1, 1), lambda i: (0, 0)),
        ],
        out_specs=pl.BlockSpec((block_r, 1), lambda i: (i, 0)),
        out_shape=jax.ShapeDtypeStruct((n, 1), jnp.float32),
        compiler_params=pltpu.CompilerParams(
            dimension_semantics=("arbitrary",),
        ),
    )(emb2d, W1, b1, W2, b2)


@jax.jit
def kernel(lp_query_doc_features, tables, W1, b1, W2, b2):
    x = lp_query_doc_features
    B, T = x.shape[0], x.shape[1]
    n_rows = B * T * F
    offs = (jnp.arange(F, dtype=jnp.int32) * VOCAB)[None, :]
    idx1 = (x.reshape(B * T, F).astype(jnp.int32) + offs).reshape(-1)
    tbl2 = tables.reshape(F * VOCAB, EDIM)
    emb = _sc_gather(idx1, tbl2, n_rows)
    emb2d = emb.reshape(B * T, F * EDIM)
    out = _mlp(emb2d, W1, b1.reshape(1, HID), W2, b2.reshape(1, 1),
               block_r=2048)
    return out.reshape(B, T)


# speedup vs baseline: 10.3210x; 1.0745x over previous
import functools

import jax
import jax.numpy as jnp
from jax import lax
from jax.experimental import pallas as pl
from jax.experimental.pallas import tpu as pltpu
from jax.experimental.pallas import tpu_sc as plsc

F, EDIM, HID = 26, 16, 32
VOCAB = 100000
NC, NS = 2, 16
NW = NC * NS
CHUNK = 128
G = 10


def _sc_gather(idx1, tbl2, n_rows):
    n_chunks = idx1.shape[0] // CHUNK
    chunks_per_w = n_chunks // NW
    groups_per_w = chunks_per_w // G
    mesh = plsc.VectorSubcoreMesh(core_axis_name="c", subcore_axis_name="s",
                                  num_cores=NC, num_subcores=NS)

    @functools.partial(
        pl.kernel,
        out_type=jax.ShapeDtypeStruct((n_rows, EDIM), jnp.float32),
        mesh=mesh,
        scratch_types=[
            pltpu.VMEM((2, G * CHUNK), jnp.int32),
            pltpu.VMEM((2, G * CHUNK, EDIM), jnp.float32),
            pltpu.SemaphoreType.DMA((2,)),
            pltpu.SemaphoreType.DMA,
        ],
        compiler_params=pltpu.CompilerParams(use_tc_tiling_on_sc=False),
    )
    def body(idx_hbm, tbl_hbm, out_hbm, idx_v, rows_v, sem_g, sem_w):
        wid = lax.axis_index("s") * NC + lax.axis_index("c")
        cbase = wid * chunks_per_w
        ng = groups_per_w

        def fire(g, p):
            rb = (cbase + g * G) * CHUNK
            pltpu.sync_copy(idx_hbm.at[pl.ds(rb, G * CHUNK)], idx_v.at[p])
            for j in range(G):
                pltpu.async_copy(
                    tbl_hbm.at[idx_v.at[p].at[pl.ds(j * CHUNK, CHUNK)]],
                    rows_v.at[p].at[pl.ds(j * CHUNK, CHUNK)],
                    sem_g.at[p],
                )

        def drain(p):
            for j in range(G):
                pltpu.make_async_copy(
                    tbl_hbm.at[pl.ds(0, CHUNK)],
                    rows_v.at[p].at[pl.ds(j * CHUNK, CHUNK)],
                    sem_g.at[p],
                ).wait()

        def wait_write(p):
            pltpu.make_async_copy(
                rows_v.at[p],
                out_hbm.at[pl.ds(0, G * CHUNK)],
                sem_w,
            ).wait()

        fire(0, 0)

        def task(g, carry):
            p = lax.rem(g, 2)
            pn = 1 - p

            @pl.when(g + 1 < ng)
            def _():
                @pl.when(g >= 1)
                def _():
                    wait_write(pn)

                fire(g + 1, pn)

            drain(p)
            rb = (cbase + g * G) * CHUNK
            pltpu.async_copy(rows_v.at[p], out_hbm.at[pl.ds(rb, G * CHUNK)],
                             sem_w)
            return carry

        lax.fori_loop(0, ng, task, 0)
        wait_write(0)
        wait_write(1)

    return body(idx1, tbl2)


def _mlp(emb2d, W1, b1, W2, b2, block_r):
    n = emb2d.shape[0]
    k = emb2d.shape[1]

    def body(emb_ref, w1_ref, b1_ref, w2_ref, b2_ref, out_ref):
        h = jnp.dot(emb_ref[...], w1_ref[...],
                    preferred_element_type=jnp.float32) + b1_ref[...]
        h = jnp.where(h > 0, h, jnp.exp(h) - 1.0)
        out_ref[...] = jnp.dot(h, w2_ref[...],
                               preferred_element_type=jnp.float32) + b2_ref[...]

    grid = (n // block_r,)
    return pl.pallas_call(
        body,
        grid=grid,
        in_specs=[
            pl.BlockSpec((block_r, k), lambda i: (i, 0)),
            pl.BlockSpec((k, HID), lambda i: (0, 0)),
            pl.BlockSpec((1, HID), lambda i: (0, 0)),
            pl.BlockSpec((HID, 1), lambda i: (0, 0)),
            pl.BlockSpec((1, 1), lambda i: (0, 0)),
        ],
        out_specs=pl.BlockSpec((block_r, 1), lambda i: (i, 0)),
        out_shape=jax.ShapeDtypeStruct((n, 1), jnp.float32),
        compiler_params=pltpu.CompilerParams(
            dimension_semantics=("arbitrary",),
        ),
    )(emb2d, W1, b1, W2, b2)


@jax.jit
def kernel(lp_query_doc_features, tables, W1, b1, W2, b2):
    x = lp_query_doc_features
    B, T = x.shape[0], x.shape[1]
    n_rows = B * T * F
    offs = (jnp.arange(F, dtype=jnp.int32) * VOCAB)[None, :]
    idx1 = (x.reshape(B * T, F).astype(jnp.int32) + offs).reshape(-1)
    tbl2 = tables.reshape(F * VOCAB, EDIM)
    emb = _sc_gather(idx1, tbl2, n_rows)
    emb2d = emb.reshape(B * T, F * EDIM)
    out = _mlp(emb2d, W1, b1.reshape(1, HID), W2, b2.reshape(1, 1),
               block_r=2048)
    return out.reshape(B, T)
